# baseline (speedup 1.0000x reference)
import functools

import jax
import jax.numpy as jnp
from jax import lax
from jax.experimental import pallas as pl
from jax.experimental.pallas import tpu as pltpu
from jax.experimental.pallas import tpu_sc as plsc

N = 10000
M = 32
D = 128
DE = 16
EPS = 1e-5

CH = 80
NCHUNK = (N * M) // CH


def _sc_gather(table, idx2):
    info = plsc.get_sparse_core_info()
    nw = info.num_cores * info.num_subcores
    chunks_per_w = NCHUNK // nw
    mesh = plsc.VectorSubcoreMesh(core_axis_name="c", subcore_axis_name="s")

    @functools.partial(
        pl.kernel,
        out_type=jax.ShapeDtypeStruct((N * M, D), jnp.float32),
        mesh=mesh,
        scratch_types=[
            pltpu.VMEM((CH,), jnp.int32),
            pltpu.VMEM((CH, D), jnp.float32),
            pltpu.SemaphoreType.DMA,
        ],
    )
    def k(table_hbm, idx_hbm, out_hbm, idx_v, rows_v, sem):
        wid = lax.axis_index("s") * info.num_cores + lax.axis_index("c")
        c0 = wid * chunks_per_w

        def body(c, _):
            row = c0 + c
            pltpu.sync_copy(idx_hbm.at[row], idx_v)
            pltpu.async_copy(table_hbm.at[idx_v], rows_v, sem).wait()
            pltpu.sync_copy(rows_v, out_hbm.at[pl.ds(row * CH, CH)])
            return _

        lax.fori_loop(0, chunks_per_w, body, 0)

    return k(table, idx2)


BN_A = 80


def _stats_kernel(a_ref, g_ref, e_ref, w_s, w_n, w_e, b_ref, sum_ref, sq_ref):
    i = pl.program_id(0)

    s = jnp.dot(a_ref[...], w_s[...], preferred_element_type=jnp.float32,
                precision=lax.Precision.HIGHEST) + b_ref[...]
    g2 = g_ref[...].reshape(BN_A * M, D)
    pg = jnp.dot(g2, w_n[...], preferred_element_type=jnp.float32,
                 precision=lax.Precision.HIGHEST).reshape(BN_A, M, 2 * D)
    e2 = e_ref[...].reshape(BN_A * M, DE)
    pe = jnp.dot(e2, w_e[...], preferred_element_type=jnp.float32,
                 precision=lax.Precision.HIGHEST).reshape(BN_A, M, 2 * D)
    x = s[:, None, :] + pg + pe

    psum = jnp.sum(x, axis=(0, 1))[None, :]
    psq = jnp.sum(x * x, axis=(0, 1))[None, :]

    @pl.when(i == 0)
    def _():
        sum_ref[...] = jnp.zeros_like(sum_ref)
        sq_ref[...] = jnp.zeros_like(sq_ref)

    sum_ref[...] += psum
    sq_ref[...] += psq


def _stats(atom, g3, nbr, w_s, w_n, w_e, b):
    grid = N // BN_A
    return pl.pallas_call(
        _stats_kernel,
        grid=(grid,),
        in_specs=[
            pl.BlockSpec((BN_A, D), lambda i: (i, 0)),
            pl.BlockSpec((BN_A, M, D), lambda i: (i, 0, 0)),
            pl.BlockSpec((BN_A, M, DE), lambda i: (i, 0, 0)),
            pl.BlockSpec((D, 2 * D), lambda i: (0, 0)),
            pl.BlockSpec((D, 2 * D), lambda i: (0, 0)),
            pl.BlockSpec((DE, 2 * D), lambda i: (0, 0)),
            pl.BlockSpec((1, 2 * D), lambda i: (0, 0)),
        ],
        out_specs=[
            pl.BlockSpec((1, 2 * D), lambda i: (0, 0)),
            pl.BlockSpec((1, 2 * D), lambda i: (0, 0)),
        ],
        out_shape=[
            jax.ShapeDtypeStruct((1, 2 * D), jnp.float32),
            jax.ShapeDtypeStruct((1, 2 * D), jnp.float32),
        ],
    )(atom, g3, nbr, w_s, w_n, w_e, b)


BN_B = 80


def _gate_kernel(a_ref, g_ref, e_ref, bw_ref, w_s, w_n, w_e, b_ref,
                 sc_ref, sh_ref, out_ref, sum2_ref, sq2_ref):
    i = pl.program_id(0)

    s = jnp.dot(a_ref[...], w_s[...], preferred_element_type=jnp.float32,
                precision=lax.Precision.HIGHEST) + b_ref[...]
    g2 = g_ref[...].reshape(BN_B * M, D)
    pg = jnp.dot(g2, w_n[...], preferred_element_type=jnp.float32,
                 precision=lax.Precision.HIGHEST).reshape(BN_B, M, 2 * D)
    e2 = e_ref[...].reshape(BN_B * M, DE)
    pe = jnp.dot(e2, w_e[...], preferred_element_type=jnp.float32,
                 precision=lax.Precision.HIGHEST).reshape(BN_B, M, 2 * D)
    x = s[:, None, :] + pg + pe
    y = x * sc_ref[...][None, :, :] + sh_ref[...][None, :, :]

    filt = jax.nn.sigmoid(y[:, :, :D])
    core = jnp.logaddexp(y[:, :, D:], 0.0)
    bw = bw_ref[...]
    prod = filt * core * (bw * bw)[:, :, None]
    ns = jnp.sum(prod, axis=1)
    out_ref[...] = ns

    @pl.when(i == 0)
    def _():
        sum2_ref[...] = jnp.zeros_like(sum2_ref)
        sq2_ref[...] = jnp.zeros_like(sq2_ref)

    sum2_ref[...] += jnp.sum(ns, axis=0)[None, :]
    sq2_ref[...] += jnp.sum(ns * ns, axis=0)[None, :]


def _gate(atom, g3, nbr, bw, w_s, w_n, w_e, b, scale1, shift1):
    grid = N // BN_B
    return pl.pallas_call(
        _gate_kernel,
        grid=(grid,),
        in_specs=[
            pl.BlockSpec((BN_B, D), lambda i: (i, 0)),
            pl.BlockSpec((BN_B, M, D), lambda i: (i, 0, 0)),
            pl.BlockSpec((BN_B, M, DE), lambda i: (i, 0, 0)),
            pl.BlockSpec((BN_B, M), lambda i: (i, 0)),
            pl.BlockSpec((D, 2 * D), lambda i: (0, 0)),
            pl.BlockSpec((D, 2 * D), lambda i: (0, 0)),
            pl.BlockSpec((DE, 2 * D), lambda i: (0, 0)),
            pl.BlockSpec((1, 2 * D), lambda i: (0, 0)),
            pl.BlockSpec((1, 2 * D), lambda i: (0, 0)),
            pl.BlockSpec((1, 2 * D), lambda i: (0, 0)),
        ],
        out_specs=[
            pl.BlockSpec((BN_B, D), lambda i: (i, 0)),
            pl.BlockSpec((1, D), lambda i: (0, 0)),
            pl.BlockSpec((1, D), lambda i: (0, 0)),
        ],
        out_shape=[
            jax.ShapeDtypeStruct((N, D), jnp.float32),
            jax.ShapeDtypeStruct((1, D), jnp.float32),
            jax.ShapeDtypeStruct((1, D), jnp.float32),
        ],
    )(atom, g3, nbr, bw, w_s, w_n, w_e, b, scale1, shift1)


BN_C = 2000


def _final_kernel(a_ref, ns_ref, sc_ref, sh_ref, out_ref):
    z = ns_ref[...] * sc_ref[...] + sh_ref[...]
    out_ref[...] = jnp.logaddexp(a_ref[...] + z, 0.0)


def _final(atom, ns, scale2, shift2):
    grid = N // BN_C
    return pl.pallas_call(
        _final_kernel,
        grid=(grid,),
        in_specs=[
            pl.BlockSpec((BN_C, D), lambda i: (i, 0)),
            pl.BlockSpec((BN_C, D), lambda i: (i, 0)),
            pl.BlockSpec((1, D), lambda i: (0, 0)),
            pl.BlockSpec((1, D), lambda i: (0, 0)),
        ],
        out_specs=pl.BlockSpec((BN_C, D), lambda i: (i, 0)),
        out_shape=jax.ShapeDtypeStruct((N, D), jnp.float32),
    )(atom, ns, scale2, shift2)


def kernel(atom_in_fea, nbr_fea, nbr_fea_idx, bond_weights, W_full, b_full,
           bn1_gamma, bn1_beta, bn2_gamma, bn2_beta):
    atom_in_fea = atom_in_fea.astype(jnp.float32)
    w_s = W_full[:D, :]
    w_n = W_full[D:2 * D, :]
    w_e = W_full[2 * D:, :]
    b2d = b_full[None, :]

    idx2 = nbr_fea_idx.astype(jnp.int32).reshape(NCHUNK, CH)
    g = _sc_gather(atom_in_fea, idx2)
    g3 = g.reshape(N, M, D)

    s1, q1 = _stats(atom_in_fea, g3, nbr_fea, w_s, w_n, w_e, b2d)
    cnt = float(N * M)
    mean1 = s1 / cnt
    var1 = jnp.maximum(q1 / cnt - mean1 * mean1, 0.0)
    scale1 = bn1_gamma[None, :] * lax.rsqrt(var1 + EPS)
    shift1 = bn1_beta[None, :] - mean1 * scale1

    ns, s2, q2 = _gate(atom_in_fea, g3, nbr_fea, bond_weights,
                       w_s, w_n, w_e, b2d, scale1, shift1)
    mean2 = s2 / float(N)
    var2 = jnp.maximum(q2 / float(N) - mean2 * mean2, 0.0)
    scale2 = bn2_gamma[None, :] * lax.rsqrt(var2 + EPS)
    shift2 = bn2_beta[None, :] - mean2 * scale2

    return _final(atom_in_fea, ns, scale2, shift2)


# speedup vs baseline: 3.3827x; 3.3827x over previous
import functools

import jax
import jax.numpy as jnp
from jax import lax
from jax.experimental import pallas as pl
from jax.experimental.pallas import tpu as pltpu
from jax.experimental.pallas import tpu_sc as plsc

N = 10000
M = 32
D = 128
DE = 16
EPS = 1e-5

CH = 80
NCHUNK = (N * M) // CH


NBUF = 5


DW = D // 2


def _sc_gather(table, idx_flat, n_edges):
    info = plsc.get_sparse_core_info()
    nw = info.num_cores * info.num_subcores
    chunks_per_w = n_edges // (nw * CH)
    mesh = plsc.VectorSubcoreMesh(core_axis_name="c", subcore_axis_name="s")

    scratch = [pltpu.VMEM((chunks_per_w * CH,), jnp.int32)]
    scratch += [pltpu.VMEM((CH, D), jnp.float32) for _ in range(NBUF)]
    scratch += [pltpu.SemaphoreType.DMA for _ in range(2 * NBUF)]

    @functools.partial(
        pl.kernel,
        out_type=jax.ShapeDtypeStruct((n_edges, D), jnp.float32),
        mesh=mesh,
        scratch_types=scratch,
    )
    def k(table_hbm, idx_hbm, out_hbm, idx_v, *bufs):
        rows = bufs[:NBUF]
        gsem = bufs[NBUF:2 * NBUF]
        osem = bufs[2 * NBUF:]
        wid = lax.axis_index("s") * info.num_cores + lax.axis_index("c")
        c0 = wid * chunks_per_w

        pltpu.sync_copy(idx_hbm.at[pl.ds(c0 * CH, chunks_per_w * CH)], idx_v)

        def islice(c):
            return idx_v.at[pl.ds(pl.multiple_of(c * CH, CH), CH)]

        for b in range(NBUF):
            pltpu.async_copy(table_hbm.at[islice(b)], rows[b], gsem[b])

        def group(g, carry):
            for b in range(NBUF):
                c = g * NBUF + b
                pltpu.make_async_copy(table_hbm.at[islice(b)], rows[b],
                                      gsem[b]).wait()
                dst = out_hbm.at[pl.ds(pl.multiple_of((c0 + c) * CH, CH), CH)]
                pltpu.async_copy(rows[b], dst, osem[b])
                pltpu.make_async_copy(rows[b], dst, osem[b]).wait()

                @pl.when(c + NBUF < chunks_per_w)
                def _():
                    pltpu.async_copy(table_hbm.at[islice(c + NBUF)],
                                     rows[b], gsem[b])
            return carry

        lax.fori_loop(0, chunks_per_w // NBUF, group, 0)

    return k(table, idx_flat)


BN_A = 400


def _stats_kernel(a_ref, g_ref, e_ref, w_s, w_n, w_e, b_ref, sum_ref, sq_ref):
    i = pl.program_id(0)

    s = jnp.dot(a_ref[...], w_s[...], preferred_element_type=jnp.float32,
                precision=lax.Precision.DEFAULT) + b_ref[...]
    g2 = g_ref[...].reshape(BN_A * M, D)
    pg = jnp.dot(g2, w_n[...], preferred_element_type=jnp.float32,
                 precision=lax.Precision.DEFAULT).reshape(BN_A, M, 2 * D)

    pe = lax.dot_general(
        e_ref[...], w_e[...], (((0,), (0,)), ((), ())),
        preferred_element_type=jnp.float32,
        precision=lax.Precision.DEFAULT).reshape(BN_A, M, 2 * D)
    x = s[:, None, :] + pg + pe

    psum = jnp.sum(x, axis=(0, 1))[None, :]
    psq = jnp.sum(x * x, axis=(0, 1))[None, :]

    @pl.when(i == 0)
    def _():
        sum_ref[...] = jnp.zeros_like(sum_ref)
        sq_ref[...] = jnp.zeros_like(sq_ref)

    sum_ref[...] += psum
    sq_ref[...] += psq


def _stats(atom, g3, nbr, w_s, w_n, w_e, b, off, nblk):
    return pl.pallas_call(
        _stats_kernel,
        grid=(nblk,),
        in_specs=[
            pl.BlockSpec((BN_A, D), lambda i: (i + off, 0)),
            pl.BlockSpec((BN_A, M, D), lambda i: (i, 0, 0)),
            pl.BlockSpec((DE, BN_A * M), lambda i: (0, i + off)),
            pl.BlockSpec((D, 2 * D), lambda i: (0, 0)),
            pl.BlockSpec((D, 2 * D), lambda i: (0, 0)),
            pl.BlockSpec((DE, 2 * D), lambda i: (0, 0)),
            pl.BlockSpec((1, 2 * D), lambda i: (0, 0)),
        ],
        out_specs=[
            pl.BlockSpec((1, 2 * D), lambda i: (0, 0)),
            pl.BlockSpec((1, 2 * D), lambda i: (0, 0)),
        ],
        out_shape=[
            jax.ShapeDtypeStruct((1, 2 * D), jnp.float32),
            jax.ShapeDtypeStruct((1, 2 * D), jnp.float32),
        ],
    )(atom, g3, nbr, w_s, w_n, w_e, b)


BN_B = 400


def _gate_kernel(a_ref, g_ref, e_ref, bw_ref, w_s, w_n, w_e, b_ref,
                 out_ref, sum2_ref, sq2_ref):
    i = pl.program_id(0)

    s = jnp.dot(a_ref[...], w_s[...], preferred_element_type=jnp.float32,
                precision=lax.Precision.DEFAULT) + b_ref[...]
    g2 = g_ref[...].reshape(BN_B * M, D)
    pg = jnp.dot(g2, w_n[...], preferred_element_type=jnp.float32,
                 precision=lax.Precision.DEFAULT).reshape(BN_B, M, 2 * D)
    pe = lax.dot_general(
        e_ref[...], w_e[...], (((0,), (0,)), ((), ())),
        preferred_element_type=jnp.float32,
        precision=lax.Precision.DEFAULT).reshape(BN_B, M, 2 * D)
    y = s[:, None, :] + pg + pe

    filt = 1.0 / (1.0 + jnp.exp(-y[:, :, :D]))
    core = jnp.log1p(jnp.exp(y[:, :, D:]))
    bw = bw_ref[...]
    prod = filt * core * (bw * bw)[:, :, None]
    ns = jnp.sum(prod, axis=1)
    out_ref[...] = ns

    @pl.when(i == 0)
    def _():
        sum2_ref[...] = jnp.zeros_like(sum2_ref)
        sq2_ref[...] = jnp.zeros_like(sq2_ref)

    sum2_ref[...] += jnp.sum(ns, axis=0)[None, :]
    sq2_ref[...] += jnp.sum(ns * ns, axis=0)[None, :]


def _gate(atom, g3, nbr, bw, w_s, w_n, w_e, b, off, nblk):
    return pl.pallas_call(
        _gate_kernel,
        grid=(nblk,),
        in_specs=[
            pl.BlockSpec((BN_B, D), lambda i: (i + off, 0)),
            pl.BlockSpec((BN_B, M, D), lambda i: (i, 0, 0)),
            pl.BlockSpec((DE, BN_B * M), lambda i: (0, i + off)),
            pl.BlockSpec((BN_B, M), lambda i: (i + off, 0)),
            pl.BlockSpec((D, 2 * D), lambda i: (0, 0)),
            pl.BlockSpec((D, 2 * D), lambda i: (0, 0)),
            pl.BlockSpec((DE, 2 * D), lambda i: (0, 0)),
            pl.BlockSpec((1, 2 * D), lambda i: (0, 0)),
        ],
        out_specs=[
            pl.BlockSpec((BN_B, D), lambda i: (i, 0)),
            pl.BlockSpec((1, D), lambda i: (0, 0)),
            pl.BlockSpec((1, D), lambda i: (0, 0)),
        ],
        out_shape=[
            jax.ShapeDtypeStruct((nblk * BN_B, D), jnp.float32),
            jax.ShapeDtypeStruct((1, D), jnp.float32),
            jax.ShapeDtypeStruct((1, D), jnp.float32),
        ],
    )(atom, g3, nbr, bw, w_s, w_n, w_e, b)


BN_C = 2000


def _final_kernel(a_ref, ns_ref, sc_ref, sh_ref, out_ref):
    z = ns_ref[...] * sc_ref[...] + sh_ref[...]
    out_ref[...] = jnp.logaddexp(a_ref[...] + z, 0.0)


def _final(atom, ns, scale2, shift2):
    grid = N // BN_C
    return pl.pallas_call(
        _final_kernel,
        grid=(grid,),
        in_specs=[
            pl.BlockSpec((BN_C, D), lambda i: (i, 0)),
            pl.BlockSpec((BN_C, D), lambda i: (i, 0)),
            pl.BlockSpec((1, D), lambda i: (0, 0)),
            pl.BlockSpec((1, D), lambda i: (0, 0)),
        ],
        out_specs=pl.BlockSpec((BN_C, D), lambda i: (i, 0)),
        out_shape=jax.ShapeDtypeStruct((N, D), jnp.float32),
    )(atom, ns, scale2, shift2)


def kernel(atom_in_fea, nbr_fea, nbr_fea_idx, bond_weights, W_full, b_full,
           bn1_gamma, bn1_beta, bn2_gamma, bn2_beta):
    atom_in_fea = atom_in_fea.astype(jnp.float32)
    w_s = W_full[:D, :]
    w_n = W_full[D:2 * D, :]
    w_e = W_full[2 * D:, :]
    b2d = b_full[None, :]

    idx_flat = nbr_fea_idx.astype(jnp.int32).reshape(N * M)
    NA1 = 4800
    g1 = _sc_gather(atom_in_fea, idx_flat[:NA1 * M], NA1 * M)
    g2 = _sc_gather(atom_in_fea, idx_flat[NA1 * M:], (N - NA1) * M)
    g3a = g1.reshape(NA1, M, D)
    g3b = g2.reshape(N - NA1, M, D)

    nbr_t = nbr_fea.reshape(N * M, DE).T

    sa, qa = _stats(atom_in_fea, g3a, nbr_t, w_s, w_n, w_e, b2d,
                    0, NA1 // BN_A)
    sb, qb = _stats(atom_in_fea, g3b, nbr_t, w_s, w_n, w_e, b2d,
                    NA1 // BN_A, (N - NA1) // BN_A)
    s1 = sa + sb
    q1 = qa + qb
    cnt = float(N * M)
    mean1 = s1 / cnt
    var1 = jnp.maximum(q1 / cnt - mean1 * mean1, 0.0)
    scale1 = bn1_gamma[None, :] * lax.rsqrt(var1 + EPS)
    shift1 = bn1_beta[None, :] - mean1 * scale1

    w_sg = w_s * scale1
    w_ng = w_n * scale1
    w_eg = w_e * scale1
    b_g = b2d * scale1 + shift1
    nsa, s2a, q2a = _gate(atom_in_fea, g3a, nbr_t, bond_weights,
                          w_sg, w_ng, w_eg, b_g,
                          0, NA1 // BN_B)
    nsb, s2b, q2b = _gate(atom_in_fea, g3b, nbr_t, bond_weights,
                          w_sg, w_ng, w_eg, b_g,
                          NA1 // BN_B, (N - NA1) // BN_B)
    ns = jnp.concatenate([nsa, nsb], axis=0)
    s2 = s2a + s2b
    q2 = q2a + q2b
    mean2 = s2 / float(N)
    var2 = jnp.maximum(q2 / float(N) - mean2 * mean2, 0.0)
    scale2 = bn2_gamma[None, :] * lax.rsqrt(var2 + EPS)
    shift2 = bn2_beta[None, :] - mean2 * scale2

    return _final(atom_in_fea, ns, scale2, shift2)
